# baseline (speedup 1.0000x reference)
import functools
import jax
import jax.numpy as jnp
from jax import lax
from jax.experimental import pallas as pl
from jax.experimental.pallas import tpu as pltpu, tpu_sc as plsc

_V = 1000
_F = 26
_NC = 2
_NS = 16
_NW = _NC * _NS
_CR = 2
_LPC = _CR * _F
_PIECES = 4


def _plant(labels_v, buf, c, val):
    iota = lax.iota(jnp.int32, 16)
    for g in range(4):
        j = g * 16 + iota
        jc = jnp.minimum(j, _LPC - 1)
        lab = plsc.load_gather(labels_v, [c * _LPC + jc])
        row = jc // _F
        col = (jc % _F) * _V + lab
        if (g + 1) * 16 <= _LPC:
            plsc.store_scatter(buf, [row, col], val)
        else:
            plsc.store_scatter(buf, [row, col], val, mask=j < _LPC)


def _sc_body(rpw, labels_hbm, out_hbm, labels_v, buf0, buf1, sem0, sem1):
    nchunk = rpw // _CR
    w = lax.axis_index("s") * _NC + lax.axis_index("c")
    base_lab = pl.multiple_of(w * (rpw * _F), 8)
    pltpu.sync_copy(labels_hbm.at[pl.ds(base_lab, rpw * _F)], labels_v)

    bufs = (buf0, buf1)
    sems = (sem0, sem1)
    ones = jnp.full((16,), 1.0, jnp.float32)
    zeros = jnp.zeros((16,), jnp.float32)

    def _z(i, carry):
        for b in range(2):
            for r in range(_CR):
                bufs[b][r, pl.ds(i * 16, 16)] = zeros
        return carry

    lax.fori_loop(0, _V * _F // 16, _z, 0)

    row0 = w * rpw

    def _fire(c, b):
        pltpu.async_copy(
            bufs[b], out_hbm.at[pl.ds(row0 + c * _CR, _CR)], sems[b]
        )

    for b in range(2):
        _plant(labels_v, bufs[b], b, ones)
        _fire(b, b)

    def _step(k, carry):
        for b in range(2):
            c = 2 * k + b
            pltpu.make_async_copy(
                bufs[b], out_hbm.at[pl.ds(row0, _CR)], sems[b]
            ).wait()
            _plant(labels_v, bufs[b], c - 2, zeros)
            _plant(labels_v, bufs[b], c, ones)
            _fire(c, b)
        return carry

    lax.fori_loop(1, nchunk // 2, _step, 0)

    for b in range(2):
        pltpu.make_async_copy(
            bufs[b], out_hbm.at[pl.ds(row0, _CR)], sems[b]
        ).wait()


def _sc_piece(pb):
    rpw = pb // _NW
    mesh = plsc.VectorSubcoreMesh(core_axis_name="c", subcore_axis_name="s")
    return pl.kernel(
        functools.partial(_sc_body, rpw),
        out_type=jax.ShapeDtypeStruct((pb, _F * _V), jnp.float32),
        mesh=mesh,
        compiler_params=pltpu.CompilerParams(needs_layout_passes=False),
        scratch_types=[
            pltpu.VMEM((rpw * _F,), jnp.int32),
            pltpu.VMEM((_CR, _F * _V), jnp.float32),
            pltpu.VMEM((_CR, _F * _V), jnp.float32),
            pltpu.SemaphoreType.DMA,
            pltpu.SemaphoreType.DMA,
        ],
    )


def kernel(labels):
    if labels.ndim == 1:
        labels = labels.reshape(labels.shape[0], -1)
    b, f = labels.shape
    pb = b // _PIECES
    run = _sc_piece(pb)
    flat = labels.reshape(b * f)
    parts = [
        run(lax.dynamic_slice_in_dim(flat, i * pb * f, pb * f))
        for i in range(_PIECES)
    ]
    return jnp.concatenate(parts, axis=0)


# speedup vs baseline: 5.7039x; 5.7039x over previous
import jax
import jax.numpy as jnp
from jax.experimental import pallas as pl
from jax.experimental.pallas import tpu as pltpu

_V = 1000


def _onehot_block(labt_ref, out_ref):
    b = labt_ref.shape[-1]
    iota = jax.lax.broadcasted_iota(jnp.int32, (_V, b), 0)
    out_ref[...] = (iota == labt_ref[0]).astype(jnp.float32)


def kernel(labels):
    if labels.ndim == 1:
        labels = labels.reshape(labels.shape[0], -1)
    b, f = labels.shape
    labt = labels.T.reshape(f, 1, b)
    out_t = pl.pallas_call(
        _onehot_block,
        grid=(f,),
        in_specs=[pl.BlockSpec((1, 1, b), lambda i: (i, 0, 0))],
        out_specs=pl.BlockSpec((_V, b), lambda i: (i, 0)),
        out_shape=jax.ShapeDtypeStruct((f * _V, b), jnp.float32),
        compiler_params=pltpu.CompilerParams(
            dimension_semantics=("arbitrary",),
            vmem_limit_bytes=100 * 1024 * 1024,
        ),
    )(labt)
    return out_t.T
